# baseline (speedup 1.0000x reference)
import functools

import jax
import jax.numpy as jnp
from jax import lax
from jax.experimental import pallas as pl
from jax.experimental.pallas import tpu as pltpu
from jax.experimental.pallas import tpu_sc as plsc

_TP = 8
_W = 128
_C_WIN = 2
_BKA = 1024
_BKB = 1024


def _transpose_in_kernel(src_ref, dst_ref):
    dst_ref[...] = src_ref[...].T


def _transpose_out_kernel(src_ref, dst_ref):
    dst_ref[0] = src_ref[0].T


def kernel(x, table):
    B, L = x.shape
    V, E = table.shape
    n_chunks = B // _TP
    N = B * L
    M = n_chunks * L

    info = plsc.get_sparse_core_info()
    n_workers = info.num_cores * info.num_subcores
    rows_per_worker = N // n_workers
    win_per_worker = rows_per_worker // _W
    chunk_rows = _C_WIN * _W
    chunks_per_worker = win_per_worker // _C_WIN

    idx = (
        x.astype(jnp.int32)
        .reshape(n_chunks, _TP, L)
        .transpose(1, 0, 2)
        .reshape(N)
    )

    table_t = jnp.swapaxes(table, 0, 1)
    table_rm = pl.pallas_call(
        _transpose_in_kernel,
        grid=(pl.cdiv(V, _BKA),),
        in_specs=[pl.BlockSpec((E, _BKA), lambda i: (0, i))],
        out_specs=pl.BlockSpec((_BKA, E), lambda i: (i, 0)),
        out_shape=jax.ShapeDtypeStruct((V, E), table.dtype),
    )(table_t)

    mesh = plsc.VectorSubcoreMesh(core_axis_name="c", subcore_axis_name="s")

    @functools.partial(
        pl.kernel,
        mesh=mesh,
        out_type=jax.ShapeDtypeStruct((N, E), table.dtype),
        scratch_types=[
            pltpu.VMEM((rows_per_worker,), jnp.int32),
            pltpu.VMEM((2, chunk_rows, E), table.dtype),
            pltpu.SemaphoreType.DMA,
            pltpu.SemaphoreType.DMA,
            pltpu.SemaphoreType.DMA,
            pltpu.SemaphoreType.DMA,
        ],
        compiler_params=pltpu.CompilerParams(use_tc_tiling_on_sc=False),
    )
    def gather_kernel(
        table_hbm, idx_hbm, out_hbm, idx_v, rows_v, g0, g1, w0, w1
    ):
        gsem = (g0, g1)
        wsem = (w0, w1)
        wid = lax.axis_index("s") * info.num_cores + lax.axis_index("c")
        base_row = wid * rows_per_worker
        pltpu.sync_copy(idx_hbm.at[pl.ds(base_row, rows_per_worker)], idx_v)

        def fire_gathers(g):
            b = g % 2
            return [
                pltpu.async_copy(
                    table_hbm.at[
                        idx_v.at[pl.ds(g * chunk_rows + w * _W, _W)]
                    ],
                    rows_v.at[b, pl.ds(w * _W, _W)],
                    gsem[b],
                )
                for w in range(_C_WIN)
            ]

        def fire_writeback(g):
            b = g % 2
            return pltpu.async_copy(
                rows_v.at[b],
                out_hbm.at[pl.ds(base_row + g * chunk_rows, chunk_rows)],
                wsem[b],
            )

        gops = {}
        wops = {}
        for g in range(chunks_per_worker + 1):
            if g < chunks_per_worker:
                if g >= 2:
                    wops[g - 2].wait()
                gops[g] = fire_gathers(g)
            if g >= 1:
                for c in gops[g - 1]:
                    c.wait()
                wops[g - 1] = fire_writeback(g - 1)
        wops[chunks_per_worker - 2].wait()
        wops[chunks_per_worker - 1].wait()

    rows = gather_kernel(table_rm, idx)

    rows3 = rows.reshape(_TP, M, E)
    out_t = pl.pallas_call(
        _transpose_out_kernel,
        grid=(_TP, M // _BKB),
        in_specs=[pl.BlockSpec((1, _BKB, E), lambda t, j: (t, j, 0))],
        out_specs=pl.BlockSpec((1, E, _BKB), lambda t, j: (t, 0, j)),
        out_shape=jax.ShapeDtypeStruct((_TP, E, M), table.dtype),
    )(rows3)
    return jnp.transpose(out_t, (0, 2, 1))


# speedup vs baseline: 1.6710x; 1.6710x over previous
import functools

import jax
import jax.numpy as jnp
from jax import lax
from jax.experimental import pallas as pl
from jax.experimental.pallas import tpu as pltpu
from jax.experimental.pallas import tpu_sc as plsc

_TP = 8
_W = 128
_C_WIN = 5


def kernel(x, table):
    B, L = x.shape
    V, E = table.shape
    n_chunks = B // _TP
    N = B * L

    info = plsc.get_sparse_core_info()
    n_workers = info.num_cores * info.num_subcores
    rows_per_worker = N // n_workers
    win_per_worker = rows_per_worker // _W
    chunk_rows = _C_WIN * _W
    chunks_per_worker = win_per_worker // _C_WIN

    idx = (
        x.astype(jnp.int32)
        .reshape(n_chunks, _TP, L)
        .transpose(1, 0, 2)
        .reshape(N)
    )

    mesh = plsc.VectorSubcoreMesh(core_axis_name="c", subcore_axis_name="s")

    @functools.partial(
        pl.kernel,
        mesh=mesh,
        out_type=jax.ShapeDtypeStruct((N, E), table.dtype),
        scratch_types=[
            pltpu.VMEM((rows_per_worker,), jnp.int32),
            pltpu.VMEM((2, chunk_rows, E), table.dtype),
            pltpu.SemaphoreType.DMA,
            pltpu.SemaphoreType.DMA,
            pltpu.SemaphoreType.DMA,
            pltpu.SemaphoreType.DMA,
        ],
        compiler_params=pltpu.CompilerParams(use_tc_tiling_on_sc=False),
    )
    def gather_kernel(
        table_hbm, idx_hbm, out_hbm, idx_v, rows_v, g0, g1, w0, w1
    ):
        gsem = (g0, g1)
        wsem = (w0, w1)
        wid = lax.axis_index("s") * info.num_cores + lax.axis_index("c")
        base_row = wid * rows_per_worker
        pltpu.sync_copy(idx_hbm.at[pl.ds(base_row, rows_per_worker)], idx_v)

        def fire_gathers(g):
            b = g % 2
            return [
                pltpu.async_copy(
                    table_hbm.at[
                        idx_v.at[pl.ds(g * chunk_rows + w * _W, _W)]
                    ],
                    rows_v.at[b, pl.ds(w * _W, _W)],
                    gsem[b],
                )
                for w in range(_C_WIN)
            ]

        def fire_writeback(g):
            b = g % 2
            return pltpu.async_copy(
                rows_v.at[b],
                out_hbm.at[pl.ds(base_row + g * chunk_rows, chunk_rows)],
                wsem[b],
            )

        gops = {}
        wops = {}
        for g in range(chunks_per_worker + 1):
            if g < chunks_per_worker:
                if g >= 2:
                    wops[g - 2].wait()
                gops[g] = fire_gathers(g)
            if g >= 1:
                for c in gops[g - 1]:
                    c.wait()
                wops[g - 1] = fire_writeback(g - 1)
        wops[chunks_per_worker - 2].wait()
        wops[chunks_per_worker - 1].wait()

    out = gather_kernel(table, idx)
    return out.reshape(_TP, n_chunks * L, E)


# speedup vs baseline: 2.1954x; 1.3138x over previous
import functools

import jax
import jax.numpy as jnp
from jax import lax
from jax.experimental import pallas as pl
from jax.experimental.pallas import tpu as pltpu
from jax.experimental.pallas import tpu_sc as plsc

_TP = 8
_CHUNK = 6400
_LANES = 16


def kernel(x, table):
    B, L = x.shape
    V, E = table.shape
    n_chunks = B // _TP
    N = B * L
    M = n_chunks * L

    info = plsc.get_sparse_core_info()
    rows_per_worker = 2
    chunks_per_t = M // _CHUNK
    n_chunk = N // _CHUNK

    idx = (
        x.astype(jnp.int32)
        .reshape(n_chunks, _TP, L)
        .transpose(1, 0, 2)
        .reshape(N)
    )

    table_t = jnp.swapaxes(table, 0, 1)

    mesh = plsc.VectorSubcoreMesh(core_axis_name="c", subcore_axis_name="s")

    @functools.partial(
        pl.kernel,
        mesh=mesh,
        out_type=jax.ShapeDtypeStruct((_TP, E, M), table.dtype),
        scratch_types=[
            pltpu.VMEM((V,), table.dtype),
            pltpu.VMEM((2, _CHUNK), jnp.int32),
            pltpu.VMEM((2, _CHUNK), table.dtype),
            pltpu.SemaphoreType.DMA,
            pltpu.SemaphoreType.DMA,
            pltpu.SemaphoreType.DMA,
            pltpu.SemaphoreType.DMA,
        ],
        compiler_params=pltpu.CompilerParams(needs_layout_passes=False),
    )
    def gather_kernel(
        tab_hbm, idx_hbm, out_hbm, row_v, idx_v, out_v, i0, i1, w0, w1
    ):
        isem = (i0, i1)
        wsem = (w0, w1)
        cid = lax.axis_index("c")
        sid = lax.axis_index("s")
        wid = sid * info.num_cores + cid

        def gather_chunk(b):
            @pl.loop(0, _CHUNK, step=8 * _LANES)
            def _(o):
                for k in range(8):
                    sl = pl.ds(o + k * _LANES, _LANES)
                    out_v[b, sl] = plsc.load_gather(row_v, [idx_v[b, sl]])

        def fire_idx(c, b):
            return pltpu.async_copy(
                idx_hbm.at[pl.ds(c * _CHUNK, _CHUNK)], idx_v.at[b], isem[b]
            )

        def wait_idx(b):
            pltpu.make_async_copy(
                idx_hbm.at[pl.ds(0, _CHUNK)], idx_v.at[b], isem[b]
            ).wait()

        def wait_out(b):
            pltpu.make_async_copy(
                out_v.at[b], out_hbm.at[0, 0, pl.ds(0, _CHUNK)], wsem[b]
            ).wait()

        def fire_out(c, b, e):
            t = c // chunks_per_t
            m0 = (c % chunks_per_t) * _CHUNK
            pltpu.async_copy(
                out_v.at[b],
                out_hbm.at[t, e, pl.ds(m0, _CHUNK)],
                wsem[b],
            )

        for e_i in range(rows_per_worker):
            e = wid * rows_per_worker + e_i
            pltpu.sync_copy(tab_hbm.at[e], row_v)
            fire_idx(0, 0)
            fire_idx(1, 1)
            for b in range(2):
                wait_idx(b)
                gather_chunk(b)
                fire_idx(b + 2, b)
                fire_out(b, b, e)

            @pl.loop(2, n_chunk - 2, step=2)
            def _(c0):
                for b in range(2):
                    c = c0 + b
                    wait_idx(b)
                    wait_out(b)
                    gather_chunk(b)
                    fire_idx(c + 2, b)
                    fire_out(c, b, e)

            for b in range(2):
                c = n_chunk - 2 + b
                wait_idx(b)
                wait_out(b)
                gather_chunk(b)
                fire_out(c, b, e)
            wait_out(0)
            wait_out(1)

    out_t = gather_kernel(table_t, idx)
    return jnp.transpose(out_t, (0, 2, 1))


# speedup vs baseline: 2.2050x; 1.0044x over previous
import functools

import jax
import jax.numpy as jnp
from jax import lax
from jax.experimental import pallas as pl
from jax.experimental.pallas import tpu as pltpu
from jax.experimental.pallas import tpu_sc as plsc

_TP = 8
_CHUNK = 6400
_LANES = 16


def kernel(x, table):
    B, L = x.shape
    V, E = table.shape
    n_chunks = B // _TP
    N = B * L
    M = n_chunks * L

    info = plsc.get_sparse_core_info()
    rows_per_worker = 2
    chunks_per_t = M // _CHUNK
    n_chunk = N // _CHUNK

    idx = (
        x.astype(jnp.int32)
        .reshape(n_chunks, _TP, L)
        .transpose(1, 0, 2)
        .reshape(N)
    )

    table_t = jnp.swapaxes(table, 0, 1)

    mesh = plsc.VectorSubcoreMesh(core_axis_name="c", subcore_axis_name="s")

    @functools.partial(
        pl.kernel,
        mesh=mesh,
        out_type=jax.ShapeDtypeStruct((_TP, E, M), table.dtype),
        scratch_types=[
            pltpu.VMEM((V,), table.dtype),
            pltpu.VMEM((2, _CHUNK), jnp.int32),
            pltpu.VMEM((2, _CHUNK), table.dtype),
            pltpu.SemaphoreType.DMA,
            pltpu.SemaphoreType.DMA,
            pltpu.SemaphoreType.DMA,
            pltpu.SemaphoreType.DMA,
        ],
        compiler_params=pltpu.CompilerParams(needs_layout_passes=False),
    )
    def gather_kernel(
        tab_hbm, idx_hbm, out_hbm, row_v, idx_v, out_v, i0, i1, w0, w1
    ):
        isem = (i0, i1)
        wsem = (w0, w1)
        cid = lax.axis_index("c")
        sid = lax.axis_index("s")
        wid = sid * info.num_cores + cid

        def gather_chunk(b):
            @pl.loop(0, _CHUNK, step=16 * _LANES)
            def _(o):
                for k in range(16):
                    sl = pl.ds(o + k * _LANES, _LANES)
                    out_v[b, sl] = plsc.load_gather(row_v, [idx_v[b, sl]])

        def fire_idx(c, b):
            return pltpu.async_copy(
                idx_hbm.at[pl.ds(c * _CHUNK, _CHUNK)], idx_v.at[b], isem[b]
            )

        def wait_idx(b):
            pltpu.make_async_copy(
                idx_hbm.at[pl.ds(0, _CHUNK)], idx_v.at[b], isem[b]
            ).wait()

        def wait_out(b):
            pltpu.make_async_copy(
                out_v.at[b], out_hbm.at[0, 0, pl.ds(0, _CHUNK)], wsem[b]
            ).wait()

        def fire_out(c, b, e):
            t = c // chunks_per_t
            m0 = (c % chunks_per_t) * _CHUNK
            pltpu.async_copy(
                out_v.at[b],
                out_hbm.at[t, e, pl.ds(m0, _CHUNK)],
                wsem[b],
            )

        for e_i in range(rows_per_worker):
            e = wid * rows_per_worker + e_i
            pltpu.sync_copy(tab_hbm.at[e], row_v)
            fire_idx(0, 0)
            fire_idx(1, 1)
            for b in range(2):
                wait_idx(b)
                gather_chunk(b)
                fire_idx(b + 2, b)
                fire_out(b, b, e)

            @pl.loop(2, n_chunk - 2, step=2)
            def _(c0):
                for b in range(2):
                    c = c0 + b
                    wait_idx(b)
                    wait_out(b)
                    gather_chunk(b)
                    fire_idx(c + 2, b)
                    fire_out(c, b, e)

            for b in range(2):
                c = n_chunk - 2 + b
                wait_idx(b)
                wait_out(b)
                gather_chunk(b)
                fire_out(c, b, e)
            wait_out(0)
            wait_out(1)

    out_t = gather_kernel(table_t, idx)
    return jnp.transpose(out_t, (0, 2, 1))
